# Initial kernel scaffold; baseline (speedup 1.0000x reference)
import jax
import jax.numpy as jnp
from jax.experimental import pallas as pl


def kernel(x_ip, x_op, x_dest, ei_ip_dest, ei_op_dest, ei_dest_ip, ei_dest_op, lin1_W, lin1_b, lin2_W, lin2_b, conv1_Wl, conv1_bl, conv1_Wr, conv2_Wl, conv2_bl, conv2_Wr, mlp_W1, mlp_b1, mlp_W2, mlp_b2):
    raise NotImplementedError("write your pallas kernel here")


# speedup vs baseline: 7.2176x; 7.2176x over previous
import functools

import jax
import jax.numpy as jnp
from jax import lax
from jax.experimental import pallas as pl
from jax.experimental.pallas import tpu as pltpu
from jax.experimental.pallas import tpu_sc as plsc

NC, NS, LANES = 2, 16, 16
NW = NC * NS
SUB = 128
KS = 5
SUPER = SUB * KS

_MESH = plsc.VectorSubcoreMesh(
    core_axis_name="c", subcore_axis_name="s", num_cores=NC, num_subcores=NS)


def _relu(x):
    return jnp.maximum(x, 0.0)


def _dot(a, b):
    return jnp.dot(a, b, preferred_element_type=jnp.float32)


def _lin1_body(x_ref, w_ref, b_ref, o_ref):
    for t in range(3):
        o_ref[t] = _relu(_dot(x_ref[t], w_ref[t]) + b_ref[t])


def _lin1(x, W, b, R):
    n = x.shape[1]
    return pl.pallas_call(
        _lin1_body,
        grid=(n // R,),
        in_specs=[
            pl.BlockSpec((3, R, x.shape[2]), lambda i: (0, i, 0)),
            pl.BlockSpec(W.shape, lambda i: (0, 0, 0)),
            pl.BlockSpec(b.shape, lambda i: (0, 0)),
        ],
        out_specs=pl.BlockSpec((3, R, W.shape[2]), lambda i: (0, i, 0)),
        out_shape=jax.ShapeDtypeStruct((3, n, W.shape[2]), jnp.float32),
    )(x, W, b)


def _conv_body(h_ref, p_ref, c_ref, wl_ref, bl_ref, wr_ref, w2_ref, b2_ref,
               o_ref, last):
    means = []
    for t in range(4):
        s = p_ref[2 * t] + p_ref[2 * t + 1]
        cnt = jnp.maximum(c_ref[2 * t] + c_ref[2 * t + 1], 1.0)
        means.append(s / cnt)
    h_ip, h_op, h_de = h_ref[0], h_ref[1], h_ref[2]
    c_de = _relu(_dot(means[0], wl_ref[0]) + bl_ref[0] + _dot(h_de, wr_ref[0])
                 + _dot(means[1], wl_ref[1]) + bl_ref[1] + _dot(h_de, wr_ref[1]))
    c_ip = _relu(_dot(means[2], wl_ref[2]) + bl_ref[2] + _dot(h_ip, wr_ref[2]))
    c_op = _relu(_dot(means[3], wl_ref[3]) + bl_ref[3] + _dot(h_op, wr_ref[3]))
    if last:
        o_ref[0] = _dot(c_ip, w2_ref[0])
        o_ref[1] = _dot(c_op, w2_ref[0])
        o_ref[2] = _dot(c_de, w2_ref[1]) + b2_ref[0]
    else:
        o_ref[0] = _relu(_dot(c_ip, w2_ref[0]) + b2_ref[0])
        o_ref[1] = _relu(_dot(c_op, w2_ref[1]) + b2_ref[1])
        o_ref[2] = _relu(_dot(c_de, w2_ref[2]) + b2_ref[2])


def _conv_stage(h, P, C, Wl, bl, Wr, W2, b2, R, last):
    n = h.shape[1]
    H = h.shape[2]
    return pl.pallas_call(
        functools.partial(_conv_body, last=last),
        grid=(n // R,),
        in_specs=[
            pl.BlockSpec((3, R, H), lambda i: (0, i, 0)),
            pl.BlockSpec((8, R, H), lambda i: (0, i, 0)),
            pl.BlockSpec((8, R, 1), lambda i: (0, i, 0)),
            pl.BlockSpec(Wl.shape, lambda i: (0, 0, 0)),
            pl.BlockSpec(bl.shape, lambda i: (0, 0)),
            pl.BlockSpec(Wr.shape, lambda i: (0, 0, 0)),
            pl.BlockSpec(W2.shape, lambda i: (0, 0, 0)),
            pl.BlockSpec(b2.shape, lambda i: (0, 0)),
        ],
        out_specs=pl.BlockSpec((3, R, H), lambda i: (0, i, 0)),
        out_shape=jax.ShapeDtypeStruct((3, n, H), jnp.float32),
    )(h, P, C, Wl, bl, Wr, W2, b2)


def _zero_slab(zrow, dst, start, nrows):
    off = 0
    while off < nrows:
        step = min(SUB, nrows - off)
        pltpu.sync_copy(zrow.at[pl.ds(0, step)], dst.at[pl.ds(start + off, step)])
        off += step


def _seg_sc(do_cnt, f_ip, f_op, f_de, s0, d0, s1, d1, s2, d2, s3, d3):
    n, H = f_ip.shape
    e = s0.shape[0] * SUB
    nsuper = e // SUPER
    rb = (n // NS) // 8 * 8
    ntail = n - NS * rb

    def body(fip, fop, fde, sr0, dr0, sr1, dr1, sr2, dr2, sr3, dr3,
             *out_and_scratch):
        if do_cnt:
            p_out, c_out = out_and_scratch[0], out_and_scratch[1]
            scratch = out_and_scratch[2:]
        else:
            p_out = out_and_scratch[0]
            c_out = None
            scratch = out_and_scratch[1:]
        acc, cnt, zrow, zcnt, onesr, sidx, didx, rows, gsem, ssem, csem = scratch
        cid = lax.axis_index("c")
        sid = lax.axis_index("s")
        wid = sid * NC + cid

        def fz(i, _):
            zrow[i // (H // LANES), pl.ds((i % (H // LANES)) * LANES, LANES)] = (
                jnp.zeros((LANES,), jnp.float32))
            return 0
        lax.fori_loop(0, SUB * (H // LANES), fz, 0)

        if do_cnt:
            def fzc(i, _):
                zcnt[i, pl.ds(0, LANES)] = jnp.zeros((LANES,), jnp.float32)
                return 0
            lax.fori_loop(0, SUB, fzc, 0)

            def fo(i, _):
                onesr[i, pl.ds(0, LANES)] = jnp.ones((LANES,), jnp.float32)
                return 0
            lax.fori_loop(0, SUB, fo, 0)

        feats = [fip, fop, fde, fde]
        srcs = [sr0, sr1, sr2, sr3]
        dsts = [dr0, dr1, dr2, dr3]
        for t in range(4):
            _zero_slab(zrow, acc, sid * rb, rb)
            if do_cnt:
                _zero_slab(zcnt, cnt, sid * rb, rb)

            @pl.when(sid == NS - 1)
            def _():
                _zero_slab(zrow, acc, NS * rb, ntail)
                if do_cnt:
                    _zero_slab(zcnt, cnt, NS * rb, ntail)
            plsc.subcore_barrier()

            nmine = (nsuper - wid + NW - 1) // NW

            def chunk_body(i, _, feat=feats[t], sref=srcs[t], dref=dsts[t]):
                c = wid + i * NW
                pltpu.sync_copy(sref.at[pl.ds(c * KS, KS)], sidx)
                pltpu.sync_copy(dref.at[pl.ds(c * KS, KS)], didx)
                gd = [pltpu.async_copy(feat.at[sidx.at[j]],
                                       rows.at[pl.ds(j * SUB, SUB)], gsem)
                      for j in range(KS)]
                for d in gd:
                    d.wait()
                sd = [pltpu.async_copy(rows.at[pl.ds(j * SUB, SUB)],
                                       acc.at[didx.at[j]], ssem, add=True)
                      for j in range(KS)]
                if do_cnt:
                    cd = [pltpu.async_copy(onesr, cnt.at[didx.at[j]], csem,
                                           add=True)
                          for j in range(KS)]
                    for d in cd:
                        d.wait()
                for d in sd:
                    d.wait()
                return 0
            lax.fori_loop(0, nmine, chunk_body, 0)
            plsc.subcore_barrier()

            pltpu.sync_copy(acc.at[pl.ds(sid * rb, rb)],
                            p_out.at[t, cid, pl.ds(sid * rb, rb)])
            if do_cnt:
                pltpu.sync_copy(cnt.at[pl.ds(sid * rb, rb)],
                                c_out.at[t, cid, pl.ds(sid * rb, rb)])

            @pl.when(sid == NS - 1)
            def _():
                pltpu.sync_copy(acc.at[pl.ds(NS * rb, ntail)],
                                p_out.at[t, cid, pl.ds(NS * rb, ntail)])
                if do_cnt:
                    pltpu.sync_copy(cnt.at[pl.ds(NS * rb, ntail)],
                                    c_out.at[t, cid, pl.ds(NS * rb, ntail)])
            plsc.subcore_barrier()

    out_type = (jax.ShapeDtypeStruct((4, NC, n, H), jnp.float32),)
    if do_cnt:
        out_type += (jax.ShapeDtypeStruct((4, NC, n, LANES), jnp.float32),)
    run = pl.kernel(
        body,
        out_type=out_type,
        mesh=_MESH,
        compiler_params=pltpu.CompilerParams(use_tc_tiling_on_sc=False),
        scratch_types=[
            pltpu.VMEM_SHARED((n, H), jnp.float32),
            pltpu.VMEM_SHARED((n, LANES), jnp.float32),
            pltpu.VMEM((SUB, H), jnp.float32),
            pltpu.VMEM((SUB, LANES), jnp.float32),
            pltpu.VMEM((SUB, LANES), jnp.float32),
            pltpu.VMEM((KS, SUB), jnp.int32),
            pltpu.VMEM((KS, SUB), jnp.int32),
            pltpu.VMEM((SUPER, H), jnp.float32),
            pltpu.SemaphoreType.DMA,
            pltpu.SemaphoreType.DMA,
            pltpu.SemaphoreType.DMA,
        ],
    )
    return run(f_ip, f_op, f_de, s0, d0, s1, d1, s2, d2, s3, d3)


def _score_sc(A_ip, A_op, B, wb, s0, d0, s1, d1):
    n, H = B.shape
    e = s0.shape[0] * SUB
    nsuper = e // SUPER
    ngrp = SUPER // LANES
    nq = H // LANES

    def body(aip, aop, btab, wbv_h, sr0, dr0, sr1, dr1, out,
             wbv, sidx, didx, rows, scores, gsem):
        cid = lax.axis_index("c")
        sid = lax.axis_index("s")
        wid = sid * NC + cid
        pltpu.sync_copy(wbv_h, wbv)
        lane = lax.iota(jnp.int32, LANES)

        tabs = [aip, aop]
        srcs = [sr0, sr1]
        dsts = [dr0, dr1]
        for t in range(2):
            nmine = (nsuper - wid + NW - 1) // NW

            def chunk_body(i, _, tab=tabs[t], sref=srcs[t], dref=dsts[t],
                           tbase=t * e):
                c = wid + i * NW
                base = c * SUPER
                pltpu.sync_copy(sref.at[pl.ds(c * KS, KS)], sidx)
                pltpu.sync_copy(dref.at[pl.ds(c * KS, KS)], didx)
                gd = [pltpu.async_copy(tab.at[sidx.at[j]],
                                       rows.at[pl.ds(j * SUB, SUB)], gsem)
                      for j in range(KS)]
                for d in gd:
                    d.wait()
                bd = [pltpu.async_copy(btab.at[didx.at[j]],
                                       rows.at[pl.ds(j * SUB, SUB)], gsem,
                                       add=True)
                      for j in range(KS)]
                for d in bd:
                    d.wait()

                wq = [wbv[pl.ds(q * LANES, LANES)] for q in range(nq)]
                b2v = wbv[pl.ds(H, LANES)]

                def grp_body(g, _):
                    def edge_body(j, sv):
                        ei = g * LANES + j
                        v = jnp.zeros((LANES,), jnp.float32)
                        for q in range(nq):
                            v = v + jnp.maximum(
                                rows[ei, pl.ds(q * LANES, LANES)], 0.0) * wq[q]
                        se = jnp.sum(v)
                        return jnp.where(lane == j, se, sv)
                    sv = lax.fori_loop(0, LANES, edge_body,
                                       jnp.zeros((LANES,), jnp.float32))
                    sv = sv + b2v
                    sa = jnp.abs(sv)
                    ex = jnp.exp(2.0 * sa)
                    tv = 1.0 - 2.0 / (ex + 1.0)
                    scores[pl.ds(g * LANES, LANES)] = jnp.sign(sv) * tv
                    return 0
                lax.fori_loop(0, ngrp, grp_body, 0)
                pltpu.sync_copy(scores, out.at[pl.ds(tbase + base, SUPER)])
                return 0
            lax.fori_loop(0, nmine, chunk_body, 0)

    run = pl.kernel(
        body,
        out_type=jax.ShapeDtypeStruct((2 * e,), jnp.float32),
        mesh=_MESH,
        compiler_params=pltpu.CompilerParams(
            use_tc_tiling_on_sc=False, needs_layout_passes=False),
        scratch_types=[
            pltpu.VMEM((H + LANES,), jnp.float32),
            pltpu.VMEM((KS, SUB), jnp.int32),
            pltpu.VMEM((KS, SUB), jnp.int32),
            pltpu.VMEM((SUPER, H), jnp.float32),
            pltpu.VMEM((SUPER,), jnp.float32),
            pltpu.SemaphoreType.DMA,
        ],
    )
    return run(A_ip, A_op, B, wb, s0, d0, s1, d1)


def kernel(x_ip, x_op, x_dest, ei_ip_dest, ei_op_dest, ei_dest_ip, ei_dest_op,
           lin1_W, lin1_b, lin2_W, lin2_b,
           conv1_Wl, conv1_bl, conv1_Wr,
           conv2_Wl, conv2_bl, conv2_Wr,
           mlp_W1, mlp_b1, mlp_W2, mlp_b2):
    n = x_ip.shape[0]
    H = lin1_W.shape[2]
    R = 2000

    s0, d0 = ei_ip_dest[0].reshape(-1, SUB), ei_ip_dest[1].reshape(-1, SUB)
    s1, d1 = ei_op_dest[0].reshape(-1, SUB), ei_op_dest[1].reshape(-1, SUB)
    s2, d2 = ei_dest_ip[0].reshape(-1, SUB), ei_dest_ip[1].reshape(-1, SUB)
    s3, d3 = ei_dest_op[0].reshape(-1, SUB), ei_dest_op[1].reshape(-1, SUB)

    x = jnp.stack([x_ip, x_op, x_dest])
    h = _lin1(x, lin1_W, lin1_b, R)

    P1, C1 = _seg_sc(True, h[0], h[1], h[2], s0, d0, s1, d1, s2, d2, s3, d3)
    C = C1[:, :, :, 0].reshape(8, n, 1)
    g = _conv_stage(h, P1.reshape(8, n, H), C,
                    conv1_Wl, conv1_bl, conv1_Wr, lin2_W, lin2_b, R, False)

    P2, = _seg_sc(False, g[0], g[1], g[2], s0, d0, s1, d1, s2, d2, s3, d3)
    W1s = jnp.stack([mlp_W1[:H], mlp_W1[H:]])
    T = _conv_stage(g, P2.reshape(8, n, H), C,
                    conv2_Wl, conv2_bl, conv2_Wr, W1s,
                    jnp.stack([mlp_b1, mlp_b1, mlp_b1]), R, True)

    wb = jnp.concatenate([mlp_W2[:, 0], jnp.broadcast_to(mlp_b2, (LANES,))])
    return _score_sc(T[0], T[1], T[2], wb, s0, d0, s1, d1)


# speedup vs baseline: 7.6920x; 1.0657x over previous
import functools

import jax
import jax.numpy as jnp
from jax import lax
from jax.experimental import pallas as pl
from jax.experimental.pallas import tpu as pltpu
from jax.experimental.pallas import tpu_sc as plsc

NC, NS, LANES = 2, 16, 16
NW = NC * NS
SUB = 128
KS = 5
SUPER = SUB * KS

_MESH = plsc.VectorSubcoreMesh(
    core_axis_name="c", subcore_axis_name="s", num_cores=NC, num_subcores=NS)


def _relu(x):
    return jnp.maximum(x, 0.0)


def _dot(a, b):
    return jnp.dot(a, b, preferred_element_type=jnp.float32)


def _lin1_body(x0_ref, x1_ref, x2_ref, w_ref, b_ref, o0_ref, o1_ref, o2_ref):
    for t, (x_ref, o_ref) in enumerate(
            [(x0_ref, o0_ref), (x1_ref, o1_ref), (x2_ref, o2_ref)]):
        o_ref[...] = _relu(_dot(x_ref[...], w_ref[t]) + b_ref[t])


def _lin1(x0, x1, x2, W, b, R):
    n, din = x0.shape
    H = W.shape[2]
    xs = pl.BlockSpec((R, din), lambda i: (i, 0))
    os = pl.BlockSpec((R, H), lambda i: (i, 0))
    oshape = jax.ShapeDtypeStruct((n, H), jnp.float32)
    return pl.pallas_call(
        _lin1_body,
        grid=(n // R,),
        in_specs=[xs, xs, xs,
                  pl.BlockSpec(W.shape, lambda i: (0, 0, 0)),
                  pl.BlockSpec(b.shape, lambda i: (0, 0))],
        out_specs=[os, os, os],
        out_shape=[oshape, oshape, oshape],
    )(x0, x1, x2, W, b)


def _conv_body(h0_ref, h1_ref, h2_ref, p_ref, c_ref, wl_ref, bl_ref, wr_ref,
               w2_ref, b2_ref, o0_ref, o1_ref, o2_ref, last):
    means = []
    for t in range(4):
        s = p_ref[2 * t] + p_ref[2 * t + 1]
        cnt = jnp.maximum(c_ref[t, 0][:, :1] + c_ref[t, 1][:, :1], 1.0)
        means.append(s / cnt)
    h_ip, h_op, h_de = h0_ref[...], h1_ref[...], h2_ref[...]
    c_de = _relu(_dot(means[0], wl_ref[0]) + bl_ref[0] + _dot(h_de, wr_ref[0])
                 + _dot(means[1], wl_ref[1]) + bl_ref[1] + _dot(h_de, wr_ref[1]))
    c_ip = _relu(_dot(means[2], wl_ref[2]) + bl_ref[2] + _dot(h_ip, wr_ref[2]))
    c_op = _relu(_dot(means[3], wl_ref[3]) + bl_ref[3] + _dot(h_op, wr_ref[3]))
    if last:
        o0_ref[...] = _dot(c_ip, w2_ref[0])
        o1_ref[...] = _dot(c_op, w2_ref[0])
        o2_ref[...] = _dot(c_de, w2_ref[1]) + b2_ref[0]
    else:
        o0_ref[...] = _relu(_dot(c_ip, w2_ref[0]) + b2_ref[0])
        o1_ref[...] = _relu(_dot(c_op, w2_ref[1]) + b2_ref[1])
        o2_ref[...] = _relu(_dot(c_de, w2_ref[2]) + b2_ref[2])


def _conv_stage(h0, h1, h2, P, C, Wl, bl, Wr, W2, b2, R, last):
    n, H = h0.shape
    hs = pl.BlockSpec((R, H), lambda i: (i, 0))
    oshape = jax.ShapeDtypeStruct((n, H), jnp.float32)
    return pl.pallas_call(
        functools.partial(_conv_body, last=last),
        grid=(n // R,),
        in_specs=[
            hs, hs, hs,
            pl.BlockSpec((8, R, H), lambda i: (0, i, 0)),
            pl.BlockSpec((4, NC, R, LANES), lambda i: (0, 0, i, 0)),
            pl.BlockSpec(Wl.shape, lambda i: (0, 0, 0)),
            pl.BlockSpec(bl.shape, lambda i: (0, 0)),
            pl.BlockSpec(Wr.shape, lambda i: (0, 0, 0)),
            pl.BlockSpec(W2.shape, lambda i: (0, 0, 0)),
            pl.BlockSpec(b2.shape, lambda i: (0, 0)),
        ],
        out_specs=[hs, hs, hs],
        out_shape=[oshape, oshape, oshape],
    )(h0, h1, h2, P, C, Wl, bl, Wr, W2, b2)


def _zero_slab(zrow, dst, start, nrows):
    off = 0
    while off < nrows:
        step = min(SUB, nrows - off)
        pltpu.sync_copy(zrow.at[pl.ds(0, step)], dst.at[pl.ds(start + off, step)])
        off += step


def _seg_sc(do_cnt, f_ip, f_op, f_de, s0, d0, s1, d1, s2, d2, s3, d3):
    n, H = f_ip.shape
    e = s0.shape[0] * SUB
    nsuper = e // SUPER
    rb = (n // NS) // 8 * 8
    ntail = n - NS * rb

    def body(fip, fop, fde, sr0, dr0, sr1, dr1, sr2, dr2, sr3, dr3,
             *out_and_scratch):
        if do_cnt:
            p_out, c_out = out_and_scratch[0], out_and_scratch[1]
            scratch = out_and_scratch[2:]
        else:
            p_out = out_and_scratch[0]
            c_out = None
            scratch = out_and_scratch[1:]
        acc, cnt, zrow, zcnt, onesr, sidx, didx, rows, gsem, ssem, csem = scratch
        cid = lax.axis_index("c")
        sid = lax.axis_index("s")
        wid = sid * NC + cid

        def fz(i, _):
            zrow[i // (H // LANES), pl.ds((i % (H // LANES)) * LANES, LANES)] = (
                jnp.zeros((LANES,), jnp.float32))
            return 0
        lax.fori_loop(0, SUB * (H // LANES), fz, 0)

        if do_cnt:
            def fzc(i, _):
                zcnt[i, pl.ds(0, LANES)] = jnp.zeros((LANES,), jnp.float32)
                return 0
            lax.fori_loop(0, SUB, fzc, 0)

            def fo(i, _):
                onesr[i, pl.ds(0, LANES)] = jnp.ones((LANES,), jnp.float32)
                return 0
            lax.fori_loop(0, SUB, fo, 0)

        feats = [fip, fop, fde, fde]
        srcs = [sr0, sr1, sr2, sr3]
        dsts = [dr0, dr1, dr2, dr3]
        for t in range(4):
            _zero_slab(zrow, acc, sid * rb, rb)
            if do_cnt:
                _zero_slab(zcnt, cnt, sid * rb, rb)

            @pl.when(sid == NS - 1)
            def _():
                _zero_slab(zrow, acc, NS * rb, ntail)
                if do_cnt:
                    _zero_slab(zcnt, cnt, NS * rb, ntail)
            plsc.subcore_barrier()

            nmine = (nsuper - wid + NW - 1) // NW

            def chunk_body(i, _, feat=feats[t], sref=srcs[t], dref=dsts[t]):
                c = wid + i * NW
                pltpu.sync_copy(sref.at[pl.ds(c * KS, KS)], sidx)
                pltpu.sync_copy(dref.at[pl.ds(c * KS, KS)], didx)
                gd = [pltpu.async_copy(feat.at[sidx.at[j]],
                                       rows.at[pl.ds(j * SUB, SUB)], gsem)
                      for j in range(KS)]
                for d in gd:
                    d.wait()
                sd = [pltpu.async_copy(rows.at[pl.ds(j * SUB, SUB)],
                                       acc.at[didx.at[j]], ssem, add=True)
                      for j in range(KS)]
                if do_cnt:
                    cd = [pltpu.async_copy(onesr, cnt.at[didx.at[j]], csem,
                                           add=True)
                          for j in range(KS)]
                    for d in cd:
                        d.wait()
                for d in sd:
                    d.wait()
                return 0
            lax.fori_loop(0, nmine, chunk_body, 0)
            plsc.subcore_barrier()

            pltpu.sync_copy(acc.at[pl.ds(sid * rb, rb)],
                            p_out.at[t, cid, pl.ds(sid * rb, rb)])
            if do_cnt:
                pltpu.sync_copy(cnt.at[pl.ds(sid * rb, rb)],
                                c_out.at[t, cid, pl.ds(sid * rb, rb)])

            @pl.when(sid == NS - 1)
            def _():
                pltpu.sync_copy(acc.at[pl.ds(NS * rb, ntail)],
                                p_out.at[t, cid, pl.ds(NS * rb, ntail)])
                if do_cnt:
                    pltpu.sync_copy(cnt.at[pl.ds(NS * rb, ntail)],
                                    c_out.at[t, cid, pl.ds(NS * rb, ntail)])
            plsc.subcore_barrier()

    out_type = (jax.ShapeDtypeStruct((4, NC, n, H), jnp.float32),)
    if do_cnt:
        out_type += (jax.ShapeDtypeStruct((4, NC, n, LANES), jnp.float32),)
    run = pl.kernel(
        body,
        out_type=out_type,
        mesh=_MESH,
        compiler_params=pltpu.CompilerParams(use_tc_tiling_on_sc=False),
        scratch_types=[
            pltpu.VMEM_SHARED((n, H), jnp.float32),
            pltpu.VMEM_SHARED((n, LANES), jnp.float32),
            pltpu.VMEM((SUB, H), jnp.float32),
            pltpu.VMEM((SUB, LANES), jnp.float32),
            pltpu.VMEM((SUB, LANES), jnp.float32),
            pltpu.VMEM((KS, SUB), jnp.int32),
            pltpu.VMEM((KS, SUB), jnp.int32),
            pltpu.VMEM((SUPER, H), jnp.float32),
            pltpu.SemaphoreType.DMA,
            pltpu.SemaphoreType.DMA,
            pltpu.SemaphoreType.DMA,
        ],
    )
    return run(f_ip, f_op, f_de, s0, d0, s1, d1, s2, d2, s3, d3)


def _score_sc(A_ip, A_op, B, wb, s0, d0, s1, d1):
    n, H = B.shape
    e = s0.shape[0] * SUB
    nsuper = e // SUPER
    ngrp = SUPER // LANES
    nq = H // LANES

    def body(aip, aop, btab, wbv_h, sr0, dr0, sr1, dr1, out,
             wbv, sidx, didx, rows, scores, gsem):
        cid = lax.axis_index("c")
        sid = lax.axis_index("s")
        wid = sid * NC + cid
        pltpu.sync_copy(wbv_h, wbv)
        lane = lax.iota(jnp.int32, LANES)

        tabs = [aip, aop]
        srcs = [sr0, sr1]
        dsts = [dr0, dr1]
        for t in range(2):
            nmine = (nsuper - wid + NW - 1) // NW

            def chunk_body(i, _, tab=tabs[t], sref=srcs[t], dref=dsts[t],
                           tbase=t * e):
                c = wid + i * NW
                base = c * SUPER
                pltpu.sync_copy(sref.at[pl.ds(c * KS, KS)], sidx)
                pltpu.sync_copy(dref.at[pl.ds(c * KS, KS)], didx)
                gd = [pltpu.async_copy(tab.at[sidx.at[j]],
                                       rows.at[pl.ds(j * SUB, SUB)], gsem)
                      for j in range(KS)]
                for d in gd:
                    d.wait()
                bd = [pltpu.async_copy(btab.at[didx.at[j]],
                                       rows.at[pl.ds(j * SUB, SUB)], gsem,
                                       add=True)
                      for j in range(KS)]
                for d in bd:
                    d.wait()

                wq = [wbv[pl.ds(q * LANES, LANES)] for q in range(nq)]
                b2v = wbv[pl.ds(H, LANES)]

                def grp_body(g, _):
                    sv = jnp.zeros((LANES,), jnp.float32)
                    for j in range(LANES):
                        ei = g * LANES + j
                        v = jnp.maximum(rows[ei, pl.ds(0, LANES)], 0.0) * wq[0]
                        for q in range(1, nq):
                            v = v + jnp.maximum(
                                rows[ei, pl.ds(q * LANES, LANES)], 0.0) * wq[q]
                        se = jnp.sum(v)
                        sv = jnp.where(lane == j, se, sv)
                    sv = sv + b2v
                    sa = jnp.abs(sv)
                    ex = jnp.exp(2.0 * sa)
                    tv = 1.0 - 2.0 / (ex + 1.0)
                    scores[pl.ds(g * LANES, LANES)] = jnp.sign(sv) * tv
                    return 0
                lax.fori_loop(0, ngrp, grp_body, 0)
                pltpu.sync_copy(scores, out.at[pl.ds(tbase + base, SUPER)])
                return 0
            lax.fori_loop(0, nmine, chunk_body, 0)

    run = pl.kernel(
        body,
        out_type=jax.ShapeDtypeStruct((2 * e,), jnp.float32),
        mesh=_MESH,
        compiler_params=pltpu.CompilerParams(
            use_tc_tiling_on_sc=False, needs_layout_passes=False),
        scratch_types=[
            pltpu.VMEM((H + LANES,), jnp.float32),
            pltpu.VMEM((KS, SUB), jnp.int32),
            pltpu.VMEM((KS, SUB), jnp.int32),
            pltpu.VMEM((SUPER, H), jnp.float32),
            pltpu.VMEM((SUPER,), jnp.float32),
            pltpu.SemaphoreType.DMA,
        ],
    )
    return run(A_ip, A_op, B, wb, s0, d0, s1, d1)


def kernel(x_ip, x_op, x_dest, ei_ip_dest, ei_op_dest, ei_dest_ip, ei_dest_op,
           lin1_W, lin1_b, lin2_W, lin2_b,
           conv1_Wl, conv1_bl, conv1_Wr,
           conv2_Wl, conv2_bl, conv2_Wr,
           mlp_W1, mlp_b1, mlp_W2, mlp_b2):
    n = x_ip.shape[0]
    H = lin1_W.shape[2]
    R = 2000

    s0, d0 = ei_ip_dest[0].reshape(-1, SUB), ei_ip_dest[1].reshape(-1, SUB)
    s1, d1 = ei_op_dest[0].reshape(-1, SUB), ei_op_dest[1].reshape(-1, SUB)
    s2, d2 = ei_dest_ip[0].reshape(-1, SUB), ei_dest_ip[1].reshape(-1, SUB)
    s3, d3 = ei_dest_op[0].reshape(-1, SUB), ei_dest_op[1].reshape(-1, SUB)

    h0, h1, h2 = _lin1(x_ip, x_op, x_dest, lin1_W, lin1_b, R)

    P1, C1 = _seg_sc(True, h0, h1, h2, s0, d0, s1, d1, s2, d2, s3, d3)
    g0, g1, g2 = _conv_stage(h0, h1, h2, P1.reshape(8, n, H), C1,
                             conv1_Wl, conv1_bl, conv1_Wr, lin2_W, lin2_b,
                             R, False)

    P2, = _seg_sc(False, g0, g1, g2, s0, d0, s1, d1, s2, d2, s3, d3)
    W1s = jnp.stack([mlp_W1[:H], mlp_W1[H:]])
    T0, T1, T2 = _conv_stage(g0, g1, g2, P2.reshape(8, n, H), C1,
                             conv2_Wl, conv2_bl, conv2_Wr, W1s,
                             jnp.stack([mlp_b1, mlp_b1, mlp_b1]), R, True)

    wb = jnp.concatenate([mlp_W2[:, 0], jnp.broadcast_to(mlp_b2, (LANES,))])
    return _score_sc(T0, T1, T2, wb, s0, d0, s1, d1)


# speedup vs baseline: 8.9412x; 1.1624x over previous
import functools

import jax
import jax.numpy as jnp
from jax import lax
from jax.experimental import pallas as pl
from jax.experimental.pallas import tpu as pltpu
from jax.experimental.pallas import tpu_sc as plsc

NC, NS, LANES = 2, 16, 16
NW = NC * NS
SUB = 128
KS = 5
SUPER = SUB * KS
ZR = 64

_MESH = plsc.VectorSubcoreMesh(
    core_axis_name="c", subcore_axis_name="s", num_cores=NC, num_subcores=NS)


def _relu(x):
    return jnp.maximum(x, 0.0)


def _dot(a, b):
    return jnp.dot(a, b, preferred_element_type=jnp.float32)


def _lin1_body(x0_ref, x1_ref, x2_ref, w_ref, b_ref, o0_ref, o1_ref, o2_ref):
    for t, (x_ref, o_ref) in enumerate(
            [(x0_ref, o0_ref), (x1_ref, o1_ref), (x2_ref, o2_ref)]):
        o_ref[...] = _relu(_dot(x_ref[...], w_ref[t]) + b_ref[t])


def _lin1(x0, x1, x2, W, b, R):
    n, din = x0.shape
    H = W.shape[2]
    xs = pl.BlockSpec((R, din), lambda i: (i, 0))
    os = pl.BlockSpec((R, H), lambda i: (i, 0))
    oshape = jax.ShapeDtypeStruct((n, H), jnp.float32)
    return pl.pallas_call(
        _lin1_body,
        grid=(n // R,),
        in_specs=[xs, xs, xs,
                  pl.BlockSpec(W.shape, lambda i: (0, 0, 0)),
                  pl.BlockSpec(b.shape, lambda i: (0, 0))],
        out_specs=[os, os, os],
        out_shape=[oshape, oshape, oshape],
    )(x0, x1, x2, W, b)


def _conv_body(h0_ref, h1_ref, h2_ref, p_ref, c_ref, wl_ref, bl_ref, wr_ref,
               w2_ref, b2_ref, o0_ref, o1_ref, o2_ref, last):
    means = []
    for t in range(4):
        s = p_ref[2 * t] + p_ref[2 * t + 1]
        cnt = jnp.maximum(c_ref[t, 0][:, :1] + c_ref[t, 1][:, :1], 1.0)
        means.append(s / cnt)
    h_ip, h_op, h_de = h0_ref[...], h1_ref[...], h2_ref[...]
    c_de = _relu(_dot(means[0], wl_ref[0]) + bl_ref[0] + _dot(h_de, wr_ref[0])
                 + _dot(means[1], wl_ref[1]) + bl_ref[1] + _dot(h_de, wr_ref[1]))
    c_ip = _relu(_dot(means[2], wl_ref[2]) + bl_ref[2] + _dot(h_ip, wr_ref[2]))
    c_op = _relu(_dot(means[3], wl_ref[3]) + bl_ref[3] + _dot(h_op, wr_ref[3]))
    if last:
        o0_ref[...] = _dot(c_ip, w2_ref[0])
        o1_ref[...] = _dot(c_op, w2_ref[0])
        o2_ref[...] = _dot(c_de, w2_ref[1]) + b2_ref[0]
    else:
        o0_ref[...] = _relu(_dot(c_ip, w2_ref[0]) + b2_ref[0])
        o1_ref[...] = _relu(_dot(c_op, w2_ref[1]) + b2_ref[1])
        o2_ref[...] = _relu(_dot(c_de, w2_ref[2]) + b2_ref[2])


def _conv_stage(h0, h1, h2, P, C, Wl, bl, Wr, W2, b2, R, last):
    n, H = h0.shape
    hs = pl.BlockSpec((R, H), lambda i: (i, 0))
    oshape = jax.ShapeDtypeStruct((n, H), jnp.float32)
    return pl.pallas_call(
        functools.partial(_conv_body, last=last),
        grid=(n // R,),
        in_specs=[
            hs, hs, hs,
            pl.BlockSpec((8, R, H), lambda i: (0, i, 0)),
            pl.BlockSpec((4, NC, R, LANES), lambda i: (0, 0, i, 0)),
            pl.BlockSpec(Wl.shape, lambda i: (0, 0, 0)),
            pl.BlockSpec(bl.shape, lambda i: (0, 0)),
            pl.BlockSpec(Wr.shape, lambda i: (0, 0, 0)),
            pl.BlockSpec(W2.shape, lambda i: (0, 0, 0)),
            pl.BlockSpec(b2.shape, lambda i: (0, 0)),
        ],
        out_specs=[hs, hs, hs],
        out_shape=[oshape, oshape, oshape],
    )(h0, h1, h2, P, C, Wl, bl, Wr, W2, b2)


def _zero_slab(zrow, dst, start, nrows):
    off = 0
    while off < nrows:
        step = min(ZR, nrows - off)
        pltpu.sync_copy(zrow.at[pl.ds(0, step)], dst.at[pl.ds(start + off, step)])
        off += step


def _fill(ref, nrows, value):
    def f(i, _):
        ref[i, pl.ds(0, LANES)] = jnp.full((LANES,), value, jnp.float32)
        return 0
    lax.fori_loop(0, nrows, f, 0)


def _seg_sc(do_cnt, f_ip, f_op, f_de, s0, d0, s1, d1, s2, d2, s3, d3):
    n, H = f_ip.shape
    e = s0.shape[0] * SUB
    nsuper = e // SUPER
    rb = (n // NS) // 8 * 8
    ntail = n - NS * rb

    def body(fip, fop, fde, sr0, dr0, sr1, dr1, sr2, dr2, sr3, dr3,
             *out_and_scratch):
        if do_cnt:
            (p_out, c_out, acc, cnt, zrow, zcnt, onesr, sidx,
             didx0, didx1, rows0, gsem, ssem, csem) = out_and_scratch
        else:
            (p_out, acc, zrow, sidx0, sidx1, didx0, didx1, rows0, rows1,
             gsem0, gsem1, ssem0, ssem1) = out_and_scratch
        cid = lax.axis_index("c")
        sid = lax.axis_index("s")
        wid = sid * NC + cid

        def fz(i, _):
            zrow[i // (H // LANES), pl.ds((i % (H // LANES)) * LANES, LANES)] = (
                jnp.zeros((LANES,), jnp.float32))
            return 0
        lax.fori_loop(0, ZR * (H // LANES), fz, 0)
        if do_cnt:
            _fill(zcnt, ZR, 0.0)
            _fill(onesr, SUB, 1.0)

        feats = [fip, fop, fde, fde]
        srcs = [sr0, sr1, sr2, sr3]
        dsts = [dr0, dr1, dr2, dr3]
        for t in range(4):
            _zero_slab(zrow, acc, sid * rb, rb)
            if do_cnt:
                _zero_slab(zcnt, cnt, sid * rb, rb)

            @pl.when(sid == NS - 1)
            def _():
                _zero_slab(zrow, acc, NS * rb, ntail)
                if do_cnt:
                    _zero_slab(zcnt, cnt, NS * rb, ntail)
            plsc.subcore_barrier()

            nmine = (nsuper - wid + NW - 1) // NW
            feat, sref, dref = feats[t], srcs[t], dsts[t]

            def drain_s(didx, ssem_):
                for j in range(KS):
                    pltpu.make_async_copy(
                        rows0.at[pl.ds(j * SUB, SUB)],
                        acc.at[didx.at[j]], ssem_).wait()

            def drain_c(didx):
                for j in range(KS):
                    pltpu.make_async_copy(onesr, cnt.at[didx.at[j]],
                                          csem).wait()

            if do_cnt:
                def half(k, par, didx, didx_o):
                    i = 2 * k + par

                    @pl.when(i < nmine)
                    def _():
                        c = wid + i * NW
                        pltpu.sync_copy(sref.at[pl.ds(c * KS, KS)], sidx)
                        pltpu.sync_copy(dref.at[pl.ds(c * KS, KS)], didx)

                        @pl.when(i >= 1)
                        def _():
                            drain_s(didx_o, ssem)
                            drain_c(didx_o)
                        gd = [pltpu.async_copy(feat.at[sidx.at[j]],
                                               rows0.at[pl.ds(j * SUB, SUB)],
                                               gsem)
                              for j in range(KS)]
                        for d in gd:
                            d.wait()
                        for j in range(KS):
                            pltpu.async_copy(rows0.at[pl.ds(j * SUB, SUB)],
                                             acc.at[didx.at[j]], ssem,
                                             add=True)
                            pltpu.async_copy(onesr, cnt.at[didx.at[j]], csem,
                                             add=True)

                def pair(k, _):
                    half(k, 0, didx0, didx1)
                    half(k, 1, didx1, didx0)
                    return 0
                lax.fori_loop(0, (nmine + 1) // 2, pair, 0)
                m = nmine - 1

                @pl.when(m % 2 == 0)
                def _():
                    drain_s(didx0, ssem)
                    drain_c(didx0)

                @pl.when(m % 2 == 1)
                def _():
                    drain_s(didx1, ssem)
                    drain_c(didx1)
            else:
                def fetch_fire(i, sidx_, didx_, rows_, gsem_):
                    c = wid + i * NW
                    pltpu.sync_copy(sref.at[pl.ds(c * KS, KS)], sidx_)
                    pltpu.sync_copy(dref.at[pl.ds(c * KS, KS)], didx_)
                    for j in range(KS):
                        pltpu.async_copy(feat.at[sidx_.at[j]],
                                         rows_.at[pl.ds(j * SUB, SUB)], gsem_)

                def drain_s2(didx_, rows_, ssem_):
                    for j in range(KS):
                        pltpu.make_async_copy(
                            rows_.at[pl.ds(j * SUB, SUB)],
                            acc.at[didx_.at[j]], ssem_).wait()

                def half(k, par, sidx_a, didx_a, rows_a, gsem_a, ssem_a,
                         sidx_b, didx_b, rows_b, gsem_b, ssem_b):
                    i = 2 * k + par

                    @pl.when(i < nmine)
                    def _():
                        for j in range(KS):
                            pltpu.make_async_copy(
                                feat.at[sidx_a.at[j]],
                                rows_a.at[pl.ds(j * SUB, SUB)], gsem_a).wait()
                        for j in range(KS):
                            pltpu.async_copy(rows_a.at[pl.ds(j * SUB, SUB)],
                                             acc.at[didx_a.at[j]], ssem_a,
                                             add=True)

                    @pl.when(i + 1 < nmine)
                    def _():
                        @pl.when(i >= 1)
                        def _():
                            drain_s2(didx_b, rows_b, ssem_b)
                        fetch_fire(i + 1, sidx_b, didx_b, rows_b, gsem_b)

                b0 = (sidx0, didx0, rows0, gsem0, ssem0)
                b1 = (sidx1, didx1, rows1, gsem1, ssem1)
                fetch_fire(0, sidx0, didx0, rows0, gsem0)

                def pair(k, _):
                    half(k, 0, *b0, *b1)
                    half(k, 1, *b1, *b0)
                    return 0
                lax.fori_loop(0, (nmine + 1) // 2, pair, 0)
                for m in (nmine - 2, nmine - 1):
                    @pl.when(jnp.logical_and(m >= 0, m % 2 == 0))
                    def _():
                        drain_s2(didx0, rows0, ssem0)

                    @pl.when(jnp.logical_and(m >= 0, m % 2 == 1))
                    def _():
                        drain_s2(didx1, rows1, ssem1)
            plsc.subcore_barrier()

            pltpu.sync_copy(acc.at[pl.ds(sid * rb, rb)],
                            p_out.at[t, cid, pl.ds(sid * rb, rb)])
            if do_cnt:
                pltpu.sync_copy(cnt.at[pl.ds(sid * rb, rb)],
                                c_out.at[t, cid, pl.ds(sid * rb, rb)])

            @pl.when(sid == NS - 1)
            def _():
                pltpu.sync_copy(acc.at[pl.ds(NS * rb, ntail)],
                                p_out.at[t, cid, pl.ds(NS * rb, ntail)])
                if do_cnt:
                    pltpu.sync_copy(cnt.at[pl.ds(NS * rb, ntail)],
                                    c_out.at[t, cid, pl.ds(NS * rb, ntail)])
            plsc.subcore_barrier()

    if do_cnt:
        out_type = (jax.ShapeDtypeStruct((4, NC, n, H), jnp.float32),
                    jax.ShapeDtypeStruct((4, NC, n, LANES), jnp.float32))
        scratch = [
            pltpu.VMEM_SHARED((n, H), jnp.float32),
            pltpu.VMEM_SHARED((n, LANES), jnp.float32),
            pltpu.VMEM((ZR, H), jnp.float32),
            pltpu.VMEM((ZR, LANES), jnp.float32),
            pltpu.VMEM((SUB, LANES), jnp.float32),
            pltpu.VMEM((KS, SUB), jnp.int32),
            pltpu.VMEM((KS, SUB), jnp.int32),
            pltpu.VMEM((KS, SUB), jnp.int32),
            pltpu.VMEM((SUPER, H), jnp.float32),
            pltpu.SemaphoreType.DMA,
            pltpu.SemaphoreType.DMA,
            pltpu.SemaphoreType.DMA,
        ]
    else:
        out_type = (jax.ShapeDtypeStruct((4, NC, n, H), jnp.float32),)
        scratch = [
            pltpu.VMEM_SHARED((n, H), jnp.float32),
            pltpu.VMEM((ZR, H), jnp.float32),
            pltpu.VMEM((KS, SUB), jnp.int32),
            pltpu.VMEM((KS, SUB), jnp.int32),
            pltpu.VMEM((KS, SUB), jnp.int32),
            pltpu.VMEM((KS, SUB), jnp.int32),
            pltpu.VMEM((SUPER, H), jnp.float32),
            pltpu.VMEM((SUPER, H), jnp.float32),
            pltpu.SemaphoreType.DMA,
            pltpu.SemaphoreType.DMA,
            pltpu.SemaphoreType.DMA,
            pltpu.SemaphoreType.DMA,
        ]
    run = pl.kernel(
        body,
        out_type=out_type,
        mesh=_MESH,
        compiler_params=pltpu.CompilerParams(use_tc_tiling_on_sc=False),
        scratch_types=scratch,
    )
    return run(f_ip, f_op, f_de, s0, d0, s1, d1, s2, d2, s3, d3)


def _score_sc(A_ip, A_op, B, wb, s0, d0, s1, d1):
    n, H = B.shape
    e = s0.shape[0] * SUB
    nsuper = e // SUPER
    ngrp = SUPER // LANES
    nq = H // LANES

    def body(aip, aop, btab, wbv_h, sr0, dr0, sr1, dr1, out,
             wbv, sidx0, sidx1, didx0, didx1, rows0, rows1, scores,
             gsem0, gsem1, bsem):
        cid = lax.axis_index("c")
        sid = lax.axis_index("s")
        wid = sid * NC + cid
        pltpu.sync_copy(wbv_h, wbv)
        lane = lax.iota(jnp.int32, LANES)

        for t, (tab, sref, dref) in enumerate(
                [(aip, sr0, dr0), (aop, sr1, dr1)]):
            tbase = t * e
            nmine = (nsuper - wid + NW - 1) // NW

            def fetch_fire(i, sidx_, didx_, rows_, gsem_):
                c = wid + i * NW
                pltpu.sync_copy(sref.at[pl.ds(c * KS, KS)], sidx_)
                pltpu.sync_copy(dref.at[pl.ds(c * KS, KS)], didx_)
                for j in range(KS):
                    pltpu.async_copy(tab.at[sidx_.at[j]],
                                     rows_.at[pl.ds(j * SUB, SUB)], gsem_)

            def compute_write(i, rows_):
                base = (wid + i * NW) * SUPER
                wq = [wbv[pl.ds(q * LANES, LANES)] for q in range(nq)]
                b2v = wbv[pl.ds(H, LANES)]

                def grp_body(g, _):
                    sv = jnp.zeros((LANES,), jnp.float32)
                    for j in range(LANES):
                        ei = g * LANES + j
                        v = jnp.maximum(rows_[ei, pl.ds(0, LANES)], 0.0) * wq[0]
                        for q in range(1, nq):
                            v = v + jnp.maximum(
                                rows_[ei, pl.ds(q * LANES, LANES)], 0.0) * wq[q]
                        se = jnp.sum(v)
                        sv = jnp.where(lane == j, se, sv)
                    sv = sv + b2v
                    sa = jnp.abs(sv)
                    ex = jnp.exp(2.0 * sa)
                    tv = 1.0 - 2.0 / (ex + 1.0)
                    scores[pl.ds(g * LANES, LANES)] = jnp.sign(sv) * tv
                    return 0
                lax.fori_loop(0, ngrp, grp_body, 0)
                pltpu.sync_copy(scores, out.at[pl.ds(tbase + base, SUPER)])

            def half(k, par, sidx_a, didx_a, rows_a, gsem_a,
                     sidx_b, didx_b, rows_b, gsem_b):
                i = 2 * k + par

                @pl.when(i < nmine)
                def _():
                    for j in range(KS):
                        pltpu.make_async_copy(
                            tab.at[sidx_a.at[j]],
                            rows_a.at[pl.ds(j * SUB, SUB)], gsem_a).wait()
                    bd = [pltpu.async_copy(btab.at[didx_a.at[j]],
                                           rows_a.at[pl.ds(j * SUB, SUB)],
                                           bsem, add=True)
                          for j in range(KS)]

                    @pl.when(i + 1 < nmine)
                    def _():
                        fetch_fire(i + 1, sidx_b, didx_b, rows_b, gsem_b)
                    for d in bd:
                        d.wait()
                    compute_write(i, rows_a)

            b0 = (sidx0, didx0, rows0, gsem0)
            b1 = (sidx1, didx1, rows1, gsem1)
            fetch_fire(0, sidx0, didx0, rows0, gsem0)

            def pair(k, _):
                half(k, 0, *b0, *b1)
                half(k, 1, *b1, *b0)
                return 0
            lax.fori_loop(0, (nmine + 1) // 2, pair, 0)

    run = pl.kernel(
        body,
        out_type=jax.ShapeDtypeStruct((2 * e,), jnp.float32),
        mesh=_MESH,
        compiler_params=pltpu.CompilerParams(
            use_tc_tiling_on_sc=False, needs_layout_passes=False),
        scratch_types=[
            pltpu.VMEM((H + LANES,), jnp.float32),
            pltpu.VMEM((KS, SUB), jnp.int32),
            pltpu.VMEM((KS, SUB), jnp.int32),
            pltpu.VMEM((KS, SUB), jnp.int32),
            pltpu.VMEM((KS, SUB), jnp.int32),
            pltpu.VMEM((SUPER, H), jnp.float32),
            pltpu.VMEM((SUPER, H), jnp.float32),
            pltpu.VMEM((SUPER,), jnp.float32),
            pltpu.SemaphoreType.DMA,
            pltpu.SemaphoreType.DMA,
            pltpu.SemaphoreType.DMA,
        ],
    )
    return run(A_ip, A_op, B, wb, s0, d0, s1, d1)


def kernel(x_ip, x_op, x_dest, ei_ip_dest, ei_op_dest, ei_dest_ip, ei_dest_op,
           lin1_W, lin1_b, lin2_W, lin2_b,
           conv1_Wl, conv1_bl, conv1_Wr,
           conv2_Wl, conv2_bl, conv2_Wr,
           mlp_W1, mlp_b1, mlp_W2, mlp_b2):
    n = x_ip.shape[0]
    H = lin1_W.shape[2]
    R = 2000

    s0, d0 = ei_ip_dest[0].reshape(-1, SUB), ei_ip_dest[1].reshape(-1, SUB)
    s1, d1 = ei_op_dest[0].reshape(-1, SUB), ei_op_dest[1].reshape(-1, SUB)
    s2, d2 = ei_dest_ip[0].reshape(-1, SUB), ei_dest_ip[1].reshape(-1, SUB)
    s3, d3 = ei_dest_op[0].reshape(-1, SUB), ei_dest_op[1].reshape(-1, SUB)

    h0, h1, h2 = _lin1(x_ip, x_op, x_dest, lin1_W, lin1_b, R)

    P1, C1 = _seg_sc(True, h0, h1, h2, s0, d0, s1, d1, s2, d2, s3, d3)
    g0, g1, g2 = _conv_stage(h0, h1, h2, P1.reshape(8, n, H), C1,
                             conv1_Wl, conv1_bl, conv1_Wr, lin2_W, lin2_b,
                             R, False)

    P2, = _seg_sc(False, g0, g1, g2, s0, d0, s1, d1, s2, d2, s3, d3)
    W1s = jnp.stack([mlp_W1[:H], mlp_W1[H:]])
    T0, T1, T2 = _conv_stage(g0, g1, g2, P2.reshape(8, n, H), C1,
                             conv2_Wl, conv2_bl, conv2_Wr, W1s,
                             jnp.stack([mlp_b1, mlp_b1, mlp_b1]), R, True)

    wb = jnp.concatenate([mlp_W2[:, 0], jnp.broadcast_to(mlp_b2, (LANES,))])
    return _score_sc(T0, T1, T2, wb, s0, d0, s1, d1)


# speedup vs baseline: 8.9719x; 1.0034x over previous
import functools

import jax
import jax.numpy as jnp
from jax import lax
from jax.experimental import pallas as pl
from jax.experimental.pallas import tpu as pltpu
from jax.experimental.pallas import tpu_sc as plsc

NC, NS, LANES = 2, 16, 16
NW = NC * NS
SUB = 128
KS = 5
SUPER = SUB * KS
ZR = 64

_MESH = plsc.VectorSubcoreMesh(
    core_axis_name="c", subcore_axis_name="s", num_cores=NC, num_subcores=NS)


def _relu(x):
    return jnp.maximum(x, 0.0)


def _dot(a, b):
    return jnp.dot(a, b, preferred_element_type=jnp.float32)


def _lin1_body(x0_ref, x1_ref, x2_ref, w_ref, b_ref, o0_ref, o1_ref, o2_ref):
    for t, (x_ref, o_ref) in enumerate(
            [(x0_ref, o0_ref), (x1_ref, o1_ref), (x2_ref, o2_ref)]):
        o_ref[...] = _relu(_dot(x_ref[...], w_ref[t]) + b_ref[t])


def _lin1(x0, x1, x2, W, b, R):
    n, din = x0.shape
    H = W.shape[2]
    xs = pl.BlockSpec((R, din), lambda i: (i, 0))
    os = pl.BlockSpec((R, H), lambda i: (i, 0))
    oshape = jax.ShapeDtypeStruct((n, H), jnp.float32)
    return pl.pallas_call(
        _lin1_body,
        grid=(n // R,),
        in_specs=[xs, xs, xs,
                  pl.BlockSpec(W.shape, lambda i: (0, 0, 0)),
                  pl.BlockSpec(b.shape, lambda i: (0, 0))],
        out_specs=[os, os, os],
        out_shape=[oshape, oshape, oshape],
    )(x0, x1, x2, W, b)


def _conv_body(h0_ref, h1_ref, h2_ref, p_ref, c_ref, wl_ref, bl_ref, wr_ref,
               w2_ref, b2_ref, o0_ref, o1_ref, o2_ref, last):
    means = []
    for t in range(4):
        s = p_ref[2 * t] + p_ref[2 * t + 1]
        cnt = jnp.maximum(c_ref[t, 0][:, :1] + c_ref[t, 1][:, :1], 1.0)
        means.append(s / cnt)
    h_ip, h_op, h_de = h0_ref[...], h1_ref[...], h2_ref[...]
    c_de = _relu(_dot(means[0], wl_ref[0]) + bl_ref[0] + _dot(h_de, wr_ref[0])
                 + _dot(means[1], wl_ref[1]) + bl_ref[1] + _dot(h_de, wr_ref[1]))
    c_ip = _relu(_dot(means[2], wl_ref[2]) + bl_ref[2] + _dot(h_ip, wr_ref[2]))
    c_op = _relu(_dot(means[3], wl_ref[3]) + bl_ref[3] + _dot(h_op, wr_ref[3]))
    if last:
        H = c_ip.shape[1]
        o0_ref[...] = _dot(c_ip, w2_ref[:H])
        o1_ref[...] = _dot(c_op, w2_ref[:H])
        o2_ref[...] = _dot(c_de, w2_ref[H:]) + b2_ref[0]
    else:
        o0_ref[...] = _relu(_dot(c_ip, w2_ref[0]) + b2_ref[0])
        o1_ref[...] = _relu(_dot(c_op, w2_ref[1]) + b2_ref[1])
        o2_ref[...] = _relu(_dot(c_de, w2_ref[2]) + b2_ref[2])


def _conv_stage(h0, h1, h2, P, C, Wl, bl, Wr, W2, b2, R, last):
    n, H = h0.shape
    hs = pl.BlockSpec((R, H), lambda i: (i, 0))
    oshape = jax.ShapeDtypeStruct((n, H), jnp.float32)
    return pl.pallas_call(
        functools.partial(_conv_body, last=last),
        grid=(n // R,),
        in_specs=[
            hs, hs, hs,
            pl.BlockSpec((8, R, H), lambda i: (0, i, 0)),
            pl.BlockSpec((4, NC, R, LANES), lambda i: (0, 0, i, 0)),
            pl.BlockSpec(Wl.shape, lambda i: (0, 0, 0)),
            pl.BlockSpec(bl.shape, lambda i: (0, 0)),
            pl.BlockSpec(Wr.shape, lambda i: (0, 0, 0)),
            pl.BlockSpec(W2.shape, lambda i: tuple([0] * len(W2.shape))),
            pl.BlockSpec(b2.shape, lambda i: (0, 0)),
        ],
        out_specs=[hs, hs, hs],
        out_shape=[oshape, oshape, oshape],
    )(h0, h1, h2, P, C, Wl, bl, Wr, W2, b2)


def _zero_slab(zrow, dst, start, nrows):
    off = 0
    while off < nrows:
        step = min(ZR, nrows - off)
        pltpu.sync_copy(zrow.at[pl.ds(0, step)], dst.at[pl.ds(start + off, step)])
        off += step


def _fill(ref, nrows, value):
    def f(i, _):
        ref[i, pl.ds(0, LANES)] = jnp.full((LANES,), value, jnp.float32)
        return 0
    lax.fori_loop(0, nrows, f, 0)


def _seg_sc(do_cnt, f_ip, f_op, f_de, s0, d0, s1, d1, s2, d2, s3, d3):
    n, H = f_ip.shape
    e = s0.shape[0] * SUB
    nsuper = e // SUPER
    rb = (n // NS) // 8 * 8
    ntail = n - NS * rb

    def body(fip, fop, fde, sr0, dr0, sr1, dr1, sr2, dr2, sr3, dr3,
             *out_and_scratch):
        if do_cnt:
            (p_out, c_out, acc, cnt, zrow, zcnt, onesr, sidx,
             didx0, didx1, rows0, gsem, ssem, csem) = out_and_scratch
        else:
            (p_out, acc, zrow, sidx0, sidx1, didx0, didx1, rows0, rows1,
             gsem0, gsem1, ssem0, ssem1) = out_and_scratch
        cid = lax.axis_index("c")
        sid = lax.axis_index("s")
        wid = sid * NC + cid

        def fz(i, _):
            zrow[i // (H // LANES), pl.ds((i % (H // LANES)) * LANES, LANES)] = (
                jnp.zeros((LANES,), jnp.float32))
            return 0
        lax.fori_loop(0, ZR * (H // LANES), fz, 0)
        if do_cnt:
            _fill(zcnt, ZR, 0.0)
            _fill(onesr, SUB, 1.0)

        feats = [fip, fop, fde, fde]
        srcs = [sr0, sr1, sr2, sr3]
        dsts = [dr0, dr1, dr2, dr3]
        for t in range(4):
            _zero_slab(zrow, acc, sid * rb, rb)
            if do_cnt:
                _zero_slab(zcnt, cnt, sid * rb, rb)

            @pl.when(sid == NS - 1)
            def _():
                _zero_slab(zrow, acc, NS * rb, ntail)
                if do_cnt:
                    _zero_slab(zcnt, cnt, NS * rb, ntail)
            plsc.subcore_barrier()

            nmine = (nsuper - wid + NW - 1) // NW
            feat, sref, dref = feats[t], srcs[t], dsts[t]

            def drain_s(didx, ssem_):
                for j in range(KS):
                    pltpu.make_async_copy(
                        rows0.at[pl.ds(j * SUB, SUB)],
                        acc.at[didx.at[j]], ssem_).wait()

            def drain_c(didx):
                for j in range(KS):
                    pltpu.make_async_copy(onesr, cnt.at[didx.at[j]],
                                          csem).wait()

            if do_cnt:
                def half(k, par, didx, didx_o):
                    i = 2 * k + par

                    @pl.when(i < nmine)
                    def _():
                        c = wid + i * NW
                        pltpu.sync_copy(sref.at[pl.ds(c * KS, KS)], sidx)
                        pltpu.sync_copy(dref.at[pl.ds(c * KS, KS)], didx)

                        @pl.when(i >= 1)
                        def _():
                            drain_s(didx_o, ssem)
                            drain_c(didx_o)
                        gd = [pltpu.async_copy(feat.at[sidx.at[j]],
                                               rows0.at[pl.ds(j * SUB, SUB)],
                                               gsem)
                              for j in range(KS)]
                        for d in gd:
                            d.wait()
                        for j in range(KS):
                            pltpu.async_copy(rows0.at[pl.ds(j * SUB, SUB)],
                                             acc.at[didx.at[j]], ssem,
                                             add=True)
                            pltpu.async_copy(onesr, cnt.at[didx.at[j]], csem,
                                             add=True)

                def pair(k, _):
                    half(k, 0, didx0, didx1)
                    half(k, 1, didx1, didx0)
                    return 0
                lax.fori_loop(0, (nmine + 1) // 2, pair, 0)
                m = nmine - 1

                @pl.when(m % 2 == 0)
                def _():
                    drain_s(didx0, ssem)
                    drain_c(didx0)

                @pl.when(m % 2 == 1)
                def _():
                    drain_s(didx1, ssem)
                    drain_c(didx1)
            else:
                def fetch_fire(i, sidx_, didx_, rows_, gsem_):
                    c = wid + i * NW
                    pltpu.sync_copy(sref.at[pl.ds(c * KS, KS)], sidx_)
                    pltpu.sync_copy(dref.at[pl.ds(c * KS, KS)], didx_)
                    for j in range(KS):
                        pltpu.async_copy(feat.at[sidx_.at[j]],
                                         rows_.at[pl.ds(j * SUB, SUB)], gsem_)

                def drain_s2(didx_, rows_, ssem_):
                    for j in range(KS):
                        pltpu.make_async_copy(
                            rows_.at[pl.ds(j * SUB, SUB)],
                            acc.at[didx_.at[j]], ssem_).wait()

                def half(k, par, sidx_a, didx_a, rows_a, gsem_a, ssem_a,
                         sidx_b, didx_b, rows_b, gsem_b, ssem_b):
                    i = 2 * k + par

                    @pl.when(i < nmine)
                    def _():
                        for j in range(KS):
                            pltpu.make_async_copy(
                                feat.at[sidx_a.at[j]],
                                rows_a.at[pl.ds(j * SUB, SUB)], gsem_a).wait()
                        for j in range(KS):
                            pltpu.async_copy(rows_a.at[pl.ds(j * SUB, SUB)],
                                             acc.at[didx_a.at[j]], ssem_a,
                                             add=True)

                    @pl.when(i + 1 < nmine)
                    def _():
                        @pl.when(i >= 1)
                        def _():
                            drain_s2(didx_b, rows_b, ssem_b)
                        fetch_fire(i + 1, sidx_b, didx_b, rows_b, gsem_b)

                b0 = (sidx0, didx0, rows0, gsem0, ssem0)
                b1 = (sidx1, didx1, rows1, gsem1, ssem1)
                fetch_fire(0, sidx0, didx0, rows0, gsem0)

                def pair(k, _):
                    half(k, 0, *b0, *b1)
                    half(k, 1, *b1, *b0)
                    return 0
                lax.fori_loop(0, (nmine + 1) // 2, pair, 0)
                for m in (nmine - 2, nmine - 1):
                    @pl.when(jnp.logical_and(m >= 0, m % 2 == 0))
                    def _():
                        drain_s2(didx0, rows0, ssem0)

                    @pl.when(jnp.logical_and(m >= 0, m % 2 == 1))
                    def _():
                        drain_s2(didx1, rows1, ssem1)
            plsc.subcore_barrier()

            pltpu.sync_copy(acc.at[pl.ds(sid * rb, rb)],
                            p_out.at[t, cid, pl.ds(sid * rb, rb)])
            if do_cnt:
                pltpu.sync_copy(cnt.at[pl.ds(sid * rb, rb)],
                                c_out.at[t, cid, pl.ds(sid * rb, rb)])

            @pl.when(sid == NS - 1)
            def _():
                pltpu.sync_copy(acc.at[pl.ds(NS * rb, ntail)],
                                p_out.at[t, cid, pl.ds(NS * rb, ntail)])
                if do_cnt:
                    pltpu.sync_copy(cnt.at[pl.ds(NS * rb, ntail)],
                                    c_out.at[t, cid, pl.ds(NS * rb, ntail)])
            plsc.subcore_barrier()

    if do_cnt:
        out_type = (jax.ShapeDtypeStruct((4, NC, n, H), jnp.float32),
                    jax.ShapeDtypeStruct((4, NC, n, LANES), jnp.float32))
        scratch = [
            pltpu.VMEM_SHARED((n, H), jnp.float32),
            pltpu.VMEM_SHARED((n, LANES), jnp.float32),
            pltpu.VMEM((ZR, H), jnp.float32),
            pltpu.VMEM((ZR, LANES), jnp.float32),
            pltpu.VMEM((SUB, LANES), jnp.float32),
            pltpu.VMEM((KS, SUB), jnp.int32),
            pltpu.VMEM((KS, SUB), jnp.int32),
            pltpu.VMEM((KS, SUB), jnp.int32),
            pltpu.VMEM((SUPER, H), jnp.float32),
            pltpu.SemaphoreType.DMA,
            pltpu.SemaphoreType.DMA,
            pltpu.SemaphoreType.DMA,
        ]
    else:
        out_type = (jax.ShapeDtypeStruct((4, NC, n, H), jnp.float32),)
        scratch = [
            pltpu.VMEM_SHARED((n, H), jnp.float32),
            pltpu.VMEM((ZR, H), jnp.float32),
            pltpu.VMEM((KS, SUB), jnp.int32),
            pltpu.VMEM((KS, SUB), jnp.int32),
            pltpu.VMEM((KS, SUB), jnp.int32),
            pltpu.VMEM((KS, SUB), jnp.int32),
            pltpu.VMEM((SUPER, H), jnp.float32),
            pltpu.VMEM((SUPER, H), jnp.float32),
            pltpu.SemaphoreType.DMA,
            pltpu.SemaphoreType.DMA,
            pltpu.SemaphoreType.DMA,
            pltpu.SemaphoreType.DMA,
        ]
    run = pl.kernel(
        body,
        out_type=out_type,
        mesh=_MESH,
        compiler_params=pltpu.CompilerParams(use_tc_tiling_on_sc=False),
        scratch_types=scratch,
    )
    return run(f_ip, f_op, f_de, s0, d0, s1, d1, s2, d2, s3, d3)


def _score_sc(A_ip, A_op, B, wb, s0, d0, s1, d1):
    n, H = B.shape
    e = s0.shape[0] * SUB
    nsuper = e // SUPER
    ngrp = SUPER // LANES
    nq = H // LANES

    def body(aip, aop, btab, wbv_h, sr0, dr0, sr1, dr1, out,
             wbv, sidx0, sidx1, didx0, didx1, rows0, rows1, scores,
             gsem0, gsem1, bsem):
        cid = lax.axis_index("c")
        sid = lax.axis_index("s")
        wid = sid * NC + cid
        pltpu.sync_copy(wbv_h, wbv)
        lane = lax.iota(jnp.int32, LANES)

        for t, (tab, sref, dref) in enumerate(
                [(aip, sr0, dr0), (aop, sr1, dr1)]):
            tbase = t * e
            nmine = (nsuper - wid + NW - 1) // NW

            def fetch_fire(i, sidx_, didx_, rows_, gsem_):
                c = wid + i * NW
                pltpu.sync_copy(sref.at[pl.ds(c * KS, KS)], sidx_)
                pltpu.sync_copy(dref.at[pl.ds(c * KS, KS)], didx_)
                for j in range(KS):
                    pltpu.async_copy(tab.at[sidx_.at[j]],
                                     rows_.at[pl.ds(j * SUB, SUB)], gsem_)

            def compute_write(i, rows_):
                base = (wid + i * NW) * SUPER
                wq = [wbv[pl.ds(q * LANES, LANES)] for q in range(nq)]
                b2v = wbv[pl.ds(H, LANES)]

                def grp_body(g, _):
                    sv = jnp.zeros((LANES,), jnp.float32)
                    for j in range(LANES):
                        ei = g * LANES + j
                        v = jnp.maximum(rows_[ei, pl.ds(0, LANES)], 0.0) * wq[0]
                        for q in range(1, nq):
                            v = v + jnp.maximum(
                                rows_[ei, pl.ds(q * LANES, LANES)], 0.0) * wq[q]
                        se = jnp.sum(v)
                        sv = jnp.where(lane == j, se, sv)
                    sv = sv + b2v
                    sa = jnp.abs(sv)
                    ex = jnp.exp(2.0 * sa)
                    tv = 1.0 - 2.0 / (ex + 1.0)
                    scores[pl.ds(g * LANES, LANES)] = jnp.sign(sv) * tv
                    return 0
                lax.fori_loop(0, ngrp, grp_body, 0)
                pltpu.sync_copy(scores, out.at[pl.ds(tbase + base, SUPER)])

            def half(k, par, sidx_a, didx_a, rows_a, gsem_a,
                     sidx_b, didx_b, rows_b, gsem_b):
                i = 2 * k + par

                @pl.when(i < nmine)
                def _():
                    for j in range(KS):
                        pltpu.make_async_copy(
                            tab.at[sidx_a.at[j]],
                            rows_a.at[pl.ds(j * SUB, SUB)], gsem_a).wait()
                    bd = [pltpu.async_copy(btab.at[didx_a.at[j]],
                                           rows_a.at[pl.ds(j * SUB, SUB)],
                                           bsem, add=True)
                          for j in range(KS)]

                    @pl.when(i + 1 < nmine)
                    def _():
                        fetch_fire(i + 1, sidx_b, didx_b, rows_b, gsem_b)
                    for d in bd:
                        d.wait()
                    compute_write(i, rows_a)

            b0 = (sidx0, didx0, rows0, gsem0)
            b1 = (sidx1, didx1, rows1, gsem1)
            fetch_fire(0, sidx0, didx0, rows0, gsem0)

            def pair(k, _):
                half(k, 0, *b0, *b1)
                half(k, 1, *b1, *b0)
                return 0
            lax.fori_loop(0, (nmine + 1) // 2, pair, 0)

    run = pl.kernel(
        body,
        out_type=jax.ShapeDtypeStruct((2 * e,), jnp.float32),
        mesh=_MESH,
        compiler_params=pltpu.CompilerParams(
            use_tc_tiling_on_sc=False, needs_layout_passes=False),
        scratch_types=[
            pltpu.VMEM((H + LANES,), jnp.float32),
            pltpu.VMEM((KS, SUB), jnp.int32),
            pltpu.VMEM((KS, SUB), jnp.int32),
            pltpu.VMEM((KS, SUB), jnp.int32),
            pltpu.VMEM((KS, SUB), jnp.int32),
            pltpu.VMEM((SUPER, H), jnp.float32),
            pltpu.VMEM((SUPER, H), jnp.float32),
            pltpu.VMEM((SUPER,), jnp.float32),
            pltpu.SemaphoreType.DMA,
            pltpu.SemaphoreType.DMA,
            pltpu.SemaphoreType.DMA,
        ],
    )
    return run(A_ip, A_op, B, wb, s0, d0, s1, d1)


def kernel(x_ip, x_op, x_dest, ei_ip_dest, ei_op_dest, ei_dest_ip, ei_dest_op,
           lin1_W, lin1_b, lin2_W, lin2_b,
           conv1_Wl, conv1_bl, conv1_Wr,
           conv2_Wl, conv2_bl, conv2_Wr,
           mlp_W1, mlp_b1, mlp_W2, mlp_b2):
    n = x_ip.shape[0]
    H = lin1_W.shape[2]
    R = 2000

    s0, d0 = ei_ip_dest[0].reshape(-1, SUB), ei_ip_dest[1].reshape(-1, SUB)
    s1, d1 = ei_op_dest[0].reshape(-1, SUB), ei_op_dest[1].reshape(-1, SUB)
    s2, d2 = ei_dest_ip[0].reshape(-1, SUB), ei_dest_ip[1].reshape(-1, SUB)
    s3, d3 = ei_dest_op[0].reshape(-1, SUB), ei_dest_op[1].reshape(-1, SUB)

    h0, h1, h2 = _lin1(x_ip, x_op, x_dest, lin1_W, lin1_b, R)

    P1, C1 = _seg_sc(True, h0, h1, h2, s0, d0, s1, d1, s2, d2, s3, d3)
    g0, g1, g2 = _conv_stage(h0, h1, h2, P1.reshape(8, n, H), C1,
                             conv1_Wl, conv1_bl, conv1_Wr, lin2_W, lin2_b,
                             R, False)

    P2, = _seg_sc(False, g0, g1, g2, s0, d0, s1, d1, s2, d2, s3, d3)
    T0, T1, T2 = _conv_stage(g0, g1, g2, P2.reshape(8, n, H), C1,
                             conv2_Wl, conv2_bl, conv2_Wr, mlp_W1,
                             mlp_b1.reshape(1, H), R, True)

    wb = jnp.concatenate([mlp_W2[:, 0], jnp.broadcast_to(mlp_b2, (LANES,))])
    return _score_sc(T0, T1, T2, wb, s0, d0, s1, d1)


# speedup vs baseline: 9.1053x; 1.0149x over previous
import functools

import jax
import jax.numpy as jnp
from jax import lax
from jax.experimental import pallas as pl
from jax.experimental.pallas import tpu as pltpu
from jax.experimental.pallas import tpu_sc as plsc

NC, NS, LANES = 2, 16, 16
NW = NC * NS
SUB = 128
KS = 5
SUPER = SUB * KS
ZR = 64

_MESH = plsc.VectorSubcoreMesh(
    core_axis_name="c", subcore_axis_name="s", num_cores=NC, num_subcores=NS)


def _relu(x):
    return jnp.maximum(x, 0.0)


def _dot(a, b):
    return jnp.dot(a, b, preferred_element_type=jnp.float32)


def _lin1_body(x0_ref, x1_ref, x2_ref, w_ref, b_ref, o0_ref, o1_ref, o2_ref):
    for t, (x_ref, o_ref) in enumerate(
            [(x0_ref, o0_ref), (x1_ref, o1_ref), (x2_ref, o2_ref)]):
        o_ref[...] = _relu(_dot(x_ref[...], w_ref[t]) + b_ref[t])


def _lin1(x0, x1, x2, W, b, R):
    n, din = x0.shape
    H = W.shape[2]
    xs = pl.BlockSpec((R, din), lambda i: (i, 0))
    os = pl.BlockSpec((R, H), lambda i: (i, 0))
    oshape = jax.ShapeDtypeStruct((n, H), jnp.float32)
    return pl.pallas_call(
        _lin1_body,
        grid=(n // R,),
        in_specs=[xs, xs, xs,
                  pl.BlockSpec(W.shape, lambda i: (0, 0, 0)),
                  pl.BlockSpec(b.shape, lambda i: (0, 0))],
        out_specs=[os, os, os],
        out_shape=[oshape, oshape, oshape],
    )(x0, x1, x2, W, b)


def _conv_body(h0_ref, h1_ref, h2_ref, p_ref, c_ref, wl_ref, bl_ref, wr_ref,
               w2_ref, b2_ref, o0_ref, o1_ref, o2_ref, last):
    means = []
    for t in range(4):
        s = p_ref[2 * t] + p_ref[2 * t + 1]
        cnt = jnp.maximum(c_ref[t, 0][:, :1] + c_ref[t, 1][:, :1], 1.0)
        means.append(s / cnt)
    h_ip, h_op, h_de = h0_ref[...], h1_ref[...], h2_ref[...]
    c_de = _relu(_dot(means[0], wl_ref[0]) + bl_ref[0] + _dot(h_de, wr_ref[0])
                 + _dot(means[1], wl_ref[1]) + bl_ref[1] + _dot(h_de, wr_ref[1]))
    c_ip = _relu(_dot(means[2], wl_ref[2]) + bl_ref[2] + _dot(h_ip, wr_ref[2]))
    c_op = _relu(_dot(means[3], wl_ref[3]) + bl_ref[3] + _dot(h_op, wr_ref[3]))
    if last:
        H = c_ip.shape[1]
        o0_ref[...] = _dot(c_ip, w2_ref[:H])
        o1_ref[...] = _dot(c_op, w2_ref[:H])
        o2_ref[...] = _dot(c_de, w2_ref[H:]) + b2_ref[0]
    else:
        o0_ref[...] = _relu(_dot(c_ip, w2_ref[0]) + b2_ref[0])
        o1_ref[...] = _relu(_dot(c_op, w2_ref[1]) + b2_ref[1])
        o2_ref[...] = _relu(_dot(c_de, w2_ref[2]) + b2_ref[2])


def _conv_stage(h0, h1, h2, P, C, Wl, bl, Wr, W2, b2, R, last):
    n, H = h0.shape
    hs = pl.BlockSpec((R, H), lambda i: (i, 0))
    oshape = jax.ShapeDtypeStruct((n, H), jnp.float32)
    return pl.pallas_call(
        functools.partial(_conv_body, last=last),
        grid=(n // R,),
        in_specs=[
            hs, hs, hs,
            pl.BlockSpec((8, R, H), lambda i: (0, i, 0)),
            pl.BlockSpec((4, NC, R, LANES), lambda i: (0, 0, i, 0)),
            pl.BlockSpec(Wl.shape, lambda i: (0, 0, 0)),
            pl.BlockSpec(bl.shape, lambda i: (0, 0)),
            pl.BlockSpec(Wr.shape, lambda i: (0, 0, 0)),
            pl.BlockSpec(W2.shape, lambda i: tuple([0] * len(W2.shape))),
            pl.BlockSpec(b2.shape, lambda i: (0, 0)),
        ],
        out_specs=[hs, hs, hs],
        out_shape=[oshape, oshape, oshape],
    )(h0, h1, h2, P, C, Wl, bl, Wr, W2, b2)


def _zero_slab(zrow, dst, start, nrows):
    off = 0
    while off < nrows:
        step = min(ZR, nrows - off)
        pltpu.sync_copy(zrow.at[pl.ds(0, step)], dst.at[pl.ds(start + off, step)])
        off += step


def _fill(ref, nrows, value):
    def f(i, _):
        ref[i, pl.ds(0, LANES)] = jnp.full((LANES,), value, jnp.float32)
        return 0
    lax.fori_loop(0, nrows, f, 0)


def _seg_sc(do_cnt, f_ip, f_op, f_de, s0, d0, s1, d1, s2, d2, s3, d3):
    n, H = f_ip.shape
    e = s0.shape[0] * SUB
    nsuper = e // SUPER
    rb = (n // NS) // 8 * 8
    ntail = n - NS * rb

    def body(fip, fop, fde, sr0, dr0, sr1, dr1, sr2, dr2, sr3, dr3,
             *out_and_scratch):
        if do_cnt:
            (p_out, c_out, acc, cnt, zrow, zcnt, onesr, sidx,
             didx0, didx1, rows0, gsem, ssem, csem) = out_and_scratch
        else:
            (p_out, acc, zrow, sidx0, sidx1, didx0, didx1, rows0, rows1,
             gsem0, gsem1, ssem0, ssem1) = out_and_scratch
        cid = lax.axis_index("c")
        sid = lax.axis_index("s")
        wid = sid * NC + cid

        def fz(i, _):
            zrow[i // (H // LANES), pl.ds((i % (H // LANES)) * LANES, LANES)] = (
                jnp.zeros((LANES,), jnp.float32))
            return 0
        lax.fori_loop(0, ZR * (H // LANES), fz, 0)
        if do_cnt:
            _fill(zcnt, ZR, 0.0)
            _fill(onesr, SUB, 1.0)

        feats = [fip, fop, fde, fde]
        srcs = [sr0, sr1, sr2, sr3]
        dsts = [dr0, dr1, dr2, dr3]
        for t in range(4):
            _zero_slab(zrow, acc, sid * rb, rb)
            if do_cnt:
                _zero_slab(zcnt, cnt, sid * rb, rb)

            @pl.when(sid == NS - 1)
            def _():
                _zero_slab(zrow, acc, NS * rb, ntail)
                if do_cnt:
                    _zero_slab(zcnt, cnt, NS * rb, ntail)
            plsc.subcore_barrier()

            nmine = (nsuper - wid + NW - 1) // NW
            feat, sref, dref = feats[t], srcs[t], dsts[t]

            def drain_s(didx, ssem_):
                for j in range(KS):
                    pltpu.make_async_copy(
                        rows0.at[pl.ds(j * SUB, SUB)],
                        acc.at[didx.at[j]], ssem_).wait()

            def drain_c(didx):
                for j in range(KS):
                    pltpu.make_async_copy(onesr, cnt.at[didx.at[j]],
                                          csem).wait()

            if do_cnt:
                def half(k, par, didx, didx_o):
                    i = 2 * k + par

                    @pl.when(i < nmine)
                    def _():
                        c = wid + i * NW
                        pltpu.sync_copy(sref.at[pl.ds(c * KS, KS)], sidx)
                        pltpu.sync_copy(dref.at[pl.ds(c * KS, KS)], didx)

                        @pl.when(i >= 1)
                        def _():
                            drain_s(didx_o, ssem)
                            drain_c(didx_o)
                        for j in range(KS):
                            pltpu.async_copy(onesr, cnt.at[didx.at[j]], csem,
                                             add=True)
                        gd = [pltpu.async_copy(feat.at[sidx.at[j]],
                                               rows0.at[pl.ds(j * SUB, SUB)],
                                               gsem)
                              for j in range(KS)]
                        for d in gd:
                            d.wait()
                        for j in range(KS):
                            pltpu.async_copy(rows0.at[pl.ds(j * SUB, SUB)],
                                             acc.at[didx.at[j]], ssem,
                                             add=True)

                def pair(k, _):
                    half(k, 0, didx0, didx1)
                    half(k, 1, didx1, didx0)
                    return 0
                lax.fori_loop(0, (nmine + 1) // 2, pair, 0)
                m = nmine - 1

                @pl.when(m % 2 == 0)
                def _():
                    drain_s(didx0, ssem)
                    drain_c(didx0)

                @pl.when(m % 2 == 1)
                def _():
                    drain_s(didx1, ssem)
                    drain_c(didx1)
            else:
                def fetch_fire(i, sidx_, didx_, rows_, gsem_):
                    c = wid + i * NW
                    pltpu.sync_copy(sref.at[pl.ds(c * KS, KS)], sidx_)
                    pltpu.sync_copy(dref.at[pl.ds(c * KS, KS)], didx_)
                    for j in range(KS):
                        pltpu.async_copy(feat.at[sidx_.at[j]],
                                         rows_.at[pl.ds(j * SUB, SUB)], gsem_)

                def drain_s2(didx_, rows_, ssem_):
                    for j in range(KS):
                        pltpu.make_async_copy(
                            rows_.at[pl.ds(j * SUB, SUB)],
                            acc.at[didx_.at[j]], ssem_).wait()

                def half(k, par, sidx_a, didx_a, rows_a, gsem_a, ssem_a,
                         sidx_b, didx_b, rows_b, gsem_b, ssem_b):
                    i = 2 * k + par

                    @pl.when(i < nmine)
                    def _():
                        for j in range(KS):
                            pltpu.make_async_copy(
                                feat.at[sidx_a.at[j]],
                                rows_a.at[pl.ds(j * SUB, SUB)], gsem_a).wait()
                        for j in range(KS):
                            pltpu.async_copy(rows_a.at[pl.ds(j * SUB, SUB)],
                                             acc.at[didx_a.at[j]], ssem_a,
                                             add=True)

                    @pl.when(i + 1 < nmine)
                    def _():
                        @pl.when(i >= 1)
                        def _():
                            drain_s2(didx_b, rows_b, ssem_b)
                        fetch_fire(i + 1, sidx_b, didx_b, rows_b, gsem_b)

                b0 = (sidx0, didx0, rows0, gsem0, ssem0)
                b1 = (sidx1, didx1, rows1, gsem1, ssem1)
                fetch_fire(0, sidx0, didx0, rows0, gsem0)

                def pair(k, _):
                    half(k, 0, *b0, *b1)
                    half(k, 1, *b1, *b0)
                    return 0
                lax.fori_loop(0, (nmine + 1) // 2, pair, 0)
                for m in (nmine - 2, nmine - 1):
                    @pl.when(jnp.logical_and(m >= 0, m % 2 == 0))
                    def _():
                        drain_s2(didx0, rows0, ssem0)

                    @pl.when(jnp.logical_and(m >= 0, m % 2 == 1))
                    def _():
                        drain_s2(didx1, rows1, ssem1)
            plsc.subcore_barrier()

            pltpu.sync_copy(acc.at[pl.ds(sid * rb, rb)],
                            p_out.at[t, cid, pl.ds(sid * rb, rb)])
            if do_cnt:
                pltpu.sync_copy(cnt.at[pl.ds(sid * rb, rb)],
                                c_out.at[t, cid, pl.ds(sid * rb, rb)])

            @pl.when(sid == NS - 1)
            def _():
                pltpu.sync_copy(acc.at[pl.ds(NS * rb, ntail)],
                                p_out.at[t, cid, pl.ds(NS * rb, ntail)])
                if do_cnt:
                    pltpu.sync_copy(cnt.at[pl.ds(NS * rb, ntail)],
                                    c_out.at[t, cid, pl.ds(NS * rb, ntail)])
            plsc.subcore_barrier()

    if do_cnt:
        out_type = (jax.ShapeDtypeStruct((4, NC, n, H), jnp.float32),
                    jax.ShapeDtypeStruct((4, NC, n, LANES), jnp.float32))
        scratch = [
            pltpu.VMEM_SHARED((n, H), jnp.float32),
            pltpu.VMEM_SHARED((n, LANES), jnp.float32),
            pltpu.VMEM((ZR, H), jnp.float32),
            pltpu.VMEM((ZR, LANES), jnp.float32),
            pltpu.VMEM((SUB, LANES), jnp.float32),
            pltpu.VMEM((KS, SUB), jnp.int32),
            pltpu.VMEM((KS, SUB), jnp.int32),
            pltpu.VMEM((KS, SUB), jnp.int32),
            pltpu.VMEM((SUPER, H), jnp.float32),
            pltpu.SemaphoreType.DMA,
            pltpu.SemaphoreType.DMA,
            pltpu.SemaphoreType.DMA,
        ]
    else:
        out_type = (jax.ShapeDtypeStruct((4, NC, n, H), jnp.float32),)
        scratch = [
            pltpu.VMEM_SHARED((n, H), jnp.float32),
            pltpu.VMEM((ZR, H), jnp.float32),
            pltpu.VMEM((KS, SUB), jnp.int32),
            pltpu.VMEM((KS, SUB), jnp.int32),
            pltpu.VMEM((KS, SUB), jnp.int32),
            pltpu.VMEM((KS, SUB), jnp.int32),
            pltpu.VMEM((SUPER, H), jnp.float32),
            pltpu.VMEM((SUPER, H), jnp.float32),
            pltpu.SemaphoreType.DMA,
            pltpu.SemaphoreType.DMA,
            pltpu.SemaphoreType.DMA,
            pltpu.SemaphoreType.DMA,
        ]
    run = pl.kernel(
        body,
        out_type=out_type,
        mesh=_MESH,
        compiler_params=pltpu.CompilerParams(use_tc_tiling_on_sc=False),
        scratch_types=scratch,
    )
    return run(f_ip, f_op, f_de, s0, d0, s1, d1, s2, d2, s3, d3)


def _score_sc(A_ip, A_op, B, wb, s0, d0, s1, d1):
    n, H = B.shape
    e = s0.shape[0] * SUB
    nsuper = e // SUPER
    ngrp = SUPER // LANES
    nq = H // LANES

    def body(aip, aop, btab, wbv_h, sr0, dr0, sr1, dr1, out,
             wbv, sidx0, sidx1, didx0, didx1, rows0, rows1, scores,
             gsem0, gsem1, bsem):
        cid = lax.axis_index("c")
        sid = lax.axis_index("s")
        wid = sid * NC + cid
        pltpu.sync_copy(wbv_h, wbv)
        lane = lax.iota(jnp.int32, LANES)

        for t, (tab, sref, dref) in enumerate(
                [(aip, sr0, dr0), (aop, sr1, dr1)]):
            tbase = t * e
            nmine = (nsuper - wid + NW - 1) // NW

            def fetch_fire(i, sidx_, didx_, rows_, gsem_):
                c = wid + i * NW
                pltpu.sync_copy(sref.at[pl.ds(c * KS, KS)], sidx_)
                pltpu.sync_copy(dref.at[pl.ds(c * KS, KS)], didx_)
                for j in range(KS):
                    pltpu.async_copy(tab.at[sidx_.at[j]],
                                     rows_.at[pl.ds(j * SUB, SUB)], gsem_)

            def compute_write(i, rows_):
                base = (wid + i * NW) * SUPER
                wq = [wbv[pl.ds(q * LANES, LANES)] for q in range(nq)]
                b2v = wbv[pl.ds(H, LANES)]

                def grp_body(g, _):
                    sv = jnp.zeros((LANES,), jnp.float32)
                    for j in range(LANES):
                        ei = g * LANES + j
                        v = jnp.maximum(rows_[ei, pl.ds(0, LANES)], 0.0) * wq[0]
                        for q in range(1, nq):
                            v = v + jnp.maximum(
                                rows_[ei, pl.ds(q * LANES, LANES)], 0.0) * wq[q]
                        se = jnp.sum(v)
                        sv = jnp.where(lane == j, se, sv)
                    sv = sv + b2v
                    sa = jnp.abs(sv)
                    ex = jnp.exp(2.0 * sa)
                    tv = 1.0 - 2.0 / (ex + 1.0)
                    scores[pl.ds(g * LANES, LANES)] = jnp.sign(sv) * tv
                    return 0
                lax.fori_loop(0, ngrp, grp_body, 0)
                pltpu.sync_copy(scores, out.at[pl.ds(tbase + base, SUPER)])

            def half(k, par, sidx_a, didx_a, rows_a, gsem_a,
                     sidx_b, didx_b, rows_b, gsem_b):
                i = 2 * k + par

                @pl.when(i < nmine)
                def _():
                    for j in range(KS):
                        pltpu.make_async_copy(
                            tab.at[sidx_a.at[j]],
                            rows_a.at[pl.ds(j * SUB, SUB)], gsem_a).wait()
                    bd = [pltpu.async_copy(btab.at[didx_a.at[j]],
                                           rows_a.at[pl.ds(j * SUB, SUB)],
                                           bsem, add=True)
                          for j in range(KS)]

                    @pl.when(i + 1 < nmine)
                    def _():
                        fetch_fire(i + 1, sidx_b, didx_b, rows_b, gsem_b)
                    for d in bd:
                        d.wait()
                    compute_write(i, rows_a)

            b0 = (sidx0, didx0, rows0, gsem0)
            b1 = (sidx1, didx1, rows1, gsem1)
            fetch_fire(0, sidx0, didx0, rows0, gsem0)

            def pair(k, _):
                half(k, 0, *b0, *b1)
                half(k, 1, *b1, *b0)
                return 0
            lax.fori_loop(0, (nmine + 1) // 2, pair, 0)

    run = pl.kernel(
        body,
        out_type=jax.ShapeDtypeStruct((2 * e,), jnp.float32),
        mesh=_MESH,
        compiler_params=pltpu.CompilerParams(
            use_tc_tiling_on_sc=False, needs_layout_passes=False),
        scratch_types=[
            pltpu.VMEM((H + LANES,), jnp.float32),
            pltpu.VMEM((KS, SUB), jnp.int32),
            pltpu.VMEM((KS, SUB), jnp.int32),
            pltpu.VMEM((KS, SUB), jnp.int32),
            pltpu.VMEM((KS, SUB), jnp.int32),
            pltpu.VMEM((SUPER, H), jnp.float32),
            pltpu.VMEM((SUPER, H), jnp.float32),
            pltpu.VMEM((SUPER,), jnp.float32),
            pltpu.SemaphoreType.DMA,
            pltpu.SemaphoreType.DMA,
            pltpu.SemaphoreType.DMA,
        ],
    )
    return run(A_ip, A_op, B, wb, s0, d0, s1, d1)


def kernel(x_ip, x_op, x_dest, ei_ip_dest, ei_op_dest, ei_dest_ip, ei_dest_op,
           lin1_W, lin1_b, lin2_W, lin2_b,
           conv1_Wl, conv1_bl, conv1_Wr,
           conv2_Wl, conv2_bl, conv2_Wr,
           mlp_W1, mlp_b1, mlp_W2, mlp_b2):
    n = x_ip.shape[0]
    H = lin1_W.shape[2]
    R = 2000

    s0, d0 = ei_ip_dest[0].reshape(-1, SUB), ei_ip_dest[1].reshape(-1, SUB)
    s1, d1 = ei_op_dest[0].reshape(-1, SUB), ei_op_dest[1].reshape(-1, SUB)
    s2, d2 = ei_dest_ip[0].reshape(-1, SUB), ei_dest_ip[1].reshape(-1, SUB)
    s3, d3 = ei_dest_op[0].reshape(-1, SUB), ei_dest_op[1].reshape(-1, SUB)

    h0, h1, h2 = _lin1(x_ip, x_op, x_dest, lin1_W, lin1_b, R)

    P1, C1 = _seg_sc(True, h0, h1, h2, s0, d0, s1, d1, s2, d2, s3, d3)
    g0, g1, g2 = _conv_stage(h0, h1, h2, P1.reshape(8, n, H), C1,
                             conv1_Wl, conv1_bl, conv1_Wr, lin2_W, lin2_b,
                             R, False)

    P2, = _seg_sc(False, g0, g1, g2, s0, d0, s1, d1, s2, d2, s3, d3)
    T0, T1, T2 = _conv_stage(g0, g1, g2, P2.reshape(8, n, H), C1,
                             conv2_Wl, conv2_bl, conv2_Wr, mlp_W1,
                             mlp_b1.reshape(1, H), R, True)

    wb = jnp.concatenate([mlp_W2[:, 0], jnp.broadcast_to(mlp_b2, (LANES,))])
    return _score_sc(T0, T1, T2, wb, s0, d0, s1, d1)
